# baseline (speedup 1.0000x reference)
import functools

import jax
import jax.numpy as jnp
from jax import lax
from jax.experimental import pallas as pl
from jax.experimental.pallas import tpu as pltpu
from jax.experimental.pallas import tpu_sc as plsc

B, C, H, W, N = 4, 96, 384, 384, 1024
HW = H * W
NC, NS, L = 2, 16, 16
CH = C // 2
NPAIR = CH // 2
NR = B * 2
RQ = H // 4
P = W
NCHUNK = RQ
NSLICE = P // L
GRP = 4

_mesh = plsc.VectorSubcoreMesh(core_axis_name="c", subcore_axis_name="s")


@functools.partial(
    pl.kernel,
    mesh=_mesh,
    compiler_params=pltpu.CompilerParams(needs_layout_passes=False),
    out_type=jax.ShapeDtypeStruct((B, C, H, W), jnp.float32),
    scratch_types=[
        pltpu.VMEM((NPAIR * N,), jnp.int32),
        pltpu.VMEM((P,), jnp.int32),
        pltpu.VMEM((P,), jnp.int32),
        pltpu.VMEM((P,), jnp.int32),
        pltpu.VMEM((CH, P), jnp.float32),
        pltpu.VMEM((CH, P), jnp.float32),
        pltpu.VMEM((CH, P), jnp.float32),
        pltpu.SemaphoreType.DMA,
        pltpu.SemaphoreType.DMA,
        pltpu.SemaphoreType.DMA,
        pltpu.SemaphoreType.DMA,
        pltpu.SemaphoreType.DMA,
        pltpu.SemaphoreType.DMA,
    ],
)
def _unpool_sc(pcb_hbm, idx_hbm, out_hbm, pcb_v, i0, i1, i2, o0, o1, o2,
               is0, is1, is2, os0, os1, os2):
    wid = lax.axis_index("s") * NC + lax.axis_index("c")
    r = wid // 4
    q = wid % 4
    b = r // 2
    coff = (r % 2) * CH
    roff = q * RQ
    idxv, outv = [i0, i1, i2], [o0, o1, o2]
    isem, osem = [is0, is1, is2], [os0, os1, os2]

    pltpu.sync_copy(pcb_hbm.at[r], pcb_v)

    def fetch_idx(g, par):
        pltpu.async_copy(idx_hbm.at[b, pl.ds((roff + g) * P, P)],
                         idxv[par], isem[par])

    def wait_idx(par):
        pltpu.make_async_copy(idx_hbm.at[b, pl.ds(0, P)],
                              idxv[par], isem[par]).wait()

    def start_out(g, par):
        pltpu.async_copy(outv[par],
                         out_hbm.at[b, pl.ds(coff, CH), roff + g],
                         osem[par])

    def wait_out(par):
        pltpu.make_async_copy(outv[par],
                              out_hbm.at[b, pl.ds(coff, CH), roff],
                              osem[par]).wait()

    def compute(par):
        iv_ref, ov = idxv[par], outv[par]
        for grp in range(NSLICE // GRP):
            ivs = tuple(iv_ref[pl.ds((grp * GRP + k) * L, L)]
                        for k in range(GRP))

            def pair_body(kp, carry, _grp=grp):
                packed = [plsc.load_gather(pcb_v, [carry[k]])
                          for k in range(GRP)]
                c2 = kp * 2
                for k in range(GRP):
                    lo, hi = plsc.unpack(
                        plsc.bitcast(packed[k], jnp.bfloat16),
                        format=plsc.PackFormat.INTERLEAVED)
                    ov[c2, pl.ds((_grp * GRP + k) * L, L)] = lo
                    ov[c2 + 1, pl.ds((_grp * GRP + k) * L, L)] = hi
                return tuple(v + N for v in carry)

            lax.fori_loop(0, NPAIR, pair_body, ivs, unroll=2)

    NBUF = 3
    for par in range(NBUF):
        fetch_idx(par, par)

    def gg_body(gg, carry):
        for par in range(NBUF):
            g = gg * NBUF + par
            wait_idx(par)

            @pl.when(gg > 0)
            def _():
                wait_out(par)

            compute(par)
            start_out(g, par)

            @pl.when(gg < NCHUNK // NBUF - 1)
            def _():
                fetch_idx(g + NBUF, par)

        return carry

    lax.fori_loop(0, NCHUNK // NBUF, gg_body, 0)
    for par in range(NBUF):
        wait_out(par)


def kernel(cat_encoded_wg, shape_input_features_in, label_mask, device):
    cb_u16 = jax.lax.bitcast_convert_type(
        cat_encoded_wg.astype(jnp.bfloat16), jnp.uint16)
    lo = cb_u16[:, 0::2, :].astype(jnp.uint32)
    hi = cb_u16[:, 1::2, :].astype(jnp.uint32)
    pcb = jax.lax.bitcast_convert_type(lo | (hi << 16), jnp.int32)
    pcb = pcb.reshape(NR, NPAIR * N)
    idx = label_mask.reshape(B, HW)
    return _unpool_sc(pcb, idx)


# speedup vs baseline: 1.0006x; 1.0006x over previous
import functools

import jax
import jax.numpy as jnp
from jax import lax
from jax.experimental import pallas as pl
from jax.experimental.pallas import tpu as pltpu
from jax.experimental.pallas import tpu_sc as plsc

B, C, H, W, N = 4, 96, 384, 384, 1024
HW = H * W
NC, NS, L = 2, 16, 16
CH = C // 2
NPAIR = CH // 2
NR = B * 2
RQ = H // 4
P = W
NCHUNK = RQ
NSLICE = P // L
GRP = 4

_mesh = plsc.VectorSubcoreMesh(core_axis_name="c", subcore_axis_name="s")


@functools.partial(
    pl.kernel,
    mesh=_mesh,
    compiler_params=pltpu.CompilerParams(needs_layout_passes=False),
    out_type=jax.ShapeDtypeStruct((B, C, H, W), jnp.float32),
    scratch_types=[
        pltpu.VMEM((NPAIR * N,), jnp.int32),
        pltpu.VMEM((P,), jnp.int32),
        pltpu.VMEM((P,), jnp.int32),
        pltpu.VMEM((CH, P), jnp.float32),
        pltpu.VMEM((CH, P), jnp.float32),
        pltpu.SemaphoreType.DMA,
        pltpu.SemaphoreType.DMA,
        pltpu.SemaphoreType.DMA,
        pltpu.SemaphoreType.DMA,
    ],
)
def _unpool_sc(pcb_hbm, idx_hbm, out_hbm, pcb_v, i0, i1, o0, o1,
               is0, is1, os0, os1):
    wid = lax.axis_index("s") * NC + lax.axis_index("c")
    r = wid // 4
    q = wid % 4
    b = r // 2
    coff = (r % 2) * CH
    roff = q * RQ
    idxv, outv = [i0, i1], [o0, o1]
    isem, osem = [is0, is1], [os0, os1]

    pltpu.sync_copy(pcb_hbm.at[r], pcb_v)

    def fetch_idx(g, par):
        pltpu.async_copy(idx_hbm.at[b, pl.ds((roff + g) * P, P)],
                         idxv[par], isem[par])

    def wait_idx(par):
        pltpu.make_async_copy(idx_hbm.at[b, pl.ds(0, P)],
                              idxv[par], isem[par]).wait()

    def start_out(g, par):
        pltpu.async_copy(outv[par],
                         out_hbm.at[b, pl.ds(coff, CH), roff + g],
                         osem[par])

    def wait_out(par):
        pltpu.make_async_copy(outv[par],
                              out_hbm.at[b, pl.ds(coff, CH), roff],
                              osem[par]).wait()

    def compute(par):
        iv_ref, ov = idxv[par], outv[par]
        for grp in range(NSLICE // GRP):
            ivs = tuple(iv_ref[pl.ds((grp * GRP + k) * L, L)]
                        for k in range(GRP))

            def pair_body(kp, carry, _grp=grp):
                packed = [plsc.load_gather(pcb_v, [carry[k]])
                          for k in range(GRP)]
                c2 = kp * 2
                for k in range(GRP):
                    lo, hi = plsc.unpack(
                        plsc.bitcast(packed[k], jnp.bfloat16),
                        format=plsc.PackFormat.INTERLEAVED)
                    ov[c2, pl.ds((_grp * GRP + k) * L, L)] = lo
                    ov[c2 + 1, pl.ds((_grp * GRP + k) * L, L)] = hi
                return tuple(v + N for v in carry)

            lax.fori_loop(0, NPAIR, pair_body, ivs, unroll=2)

    fetch_idx(0, 0)
    fetch_idx(1, 1)

    def gg_body(gg, carry):
        for par in range(2):
            g = gg * 2 + par
            wait_idx(par)

            @pl.when(gg > 0)
            def _():
                wait_out(par)

            compute(par)
            start_out(g, par)

            @pl.when(gg < NCHUNK // 2 - 1)
            def _():
                fetch_idx(g + 2, par)

        return carry

    lax.fori_loop(0, NCHUNK // 2, gg_body, 0)
    wait_out(0)
    wait_out(1)


def kernel(cat_encoded_wg, shape_input_features_in, label_mask, device):
    cb_u16 = jax.lax.bitcast_convert_type(
        cat_encoded_wg.astype(jnp.bfloat16), jnp.uint16)
    lo = cb_u16[:, 0::2, :].astype(jnp.uint32)
    hi = cb_u16[:, 1::2, :].astype(jnp.uint32)
    pcb = jax.lax.bitcast_convert_type(lo | (hi << 16), jnp.int32)
    pcb = pcb.reshape(NR, NPAIR * N)
    idx = label_mask.reshape(B, HW)
    return _unpool_sc(pcb, idx)


# speedup vs baseline: 1.1343x; 1.1336x over previous
import functools

import jax
import jax.numpy as jnp
from jax import lax
from jax.experimental import pallas as pl
from jax.experimental.pallas import tpu as pltpu
from jax.experimental.pallas import tpu_sc as plsc

B, C, H, W, N = 4, 96, 384, 384, 1024
HW = H * W
NC, NS, L = 2, 16, 16
CH = C // 2
NPAIR = CH // 2
NR = B * 2
RQ = H // 4
P = W
NCHUNK = RQ
NSLICE = P // L
GRP = 6

_mesh = plsc.VectorSubcoreMesh(core_axis_name="c", subcore_axis_name="s")


@functools.partial(
    pl.kernel,
    mesh=_mesh,
    compiler_params=pltpu.CompilerParams(needs_layout_passes=False),
    out_type=jax.ShapeDtypeStruct((B, C, H, W), jnp.float32),
    scratch_types=[
        pltpu.VMEM((NPAIR * N,), jnp.int32),
        pltpu.VMEM((P,), jnp.int32),
        pltpu.VMEM((P,), jnp.int32),
        pltpu.VMEM((CH, P), jnp.float32),
        pltpu.VMEM((CH, P), jnp.float32),
        pltpu.SemaphoreType.DMA,
        pltpu.SemaphoreType.DMA,
        pltpu.SemaphoreType.DMA,
        pltpu.SemaphoreType.DMA,
    ],
)
def _unpool_sc(pcb_hbm, idx_hbm, out_hbm, pcb_v, i0, i1, o0, o1,
               is0, is1, os0, os1):
    wid = lax.axis_index("s") * NC + lax.axis_index("c")
    r = wid // 4
    q = wid % 4
    b = r // 2
    coff = (r % 2) * CH
    roff = q * RQ
    idxv, outv = [i0, i1], [o0, o1]
    isem, osem = [is0, is1], [os0, os1]

    pltpu.sync_copy(pcb_hbm.at[r], pcb_v)

    def fetch_idx(g, par):
        pltpu.async_copy(idx_hbm.at[b, pl.ds((roff + g) * P, P)],
                         idxv[par], isem[par])

    def wait_idx(par):
        pltpu.make_async_copy(idx_hbm.at[b, pl.ds(0, P)],
                              idxv[par], isem[par]).wait()

    def start_out(g, par):
        pltpu.async_copy(outv[par],
                         out_hbm.at[b, pl.ds(coff, CH), roff + g],
                         osem[par])

    def wait_out(par):
        pltpu.make_async_copy(outv[par],
                              out_hbm.at[b, pl.ds(coff, CH), roff],
                              osem[par]).wait()

    def compute(par):
        iv_ref, ov = idxv[par], outv[par]
        for grp in range(NSLICE // GRP):
            ivs = tuple(iv_ref[pl.ds((grp * GRP + k) * L, L)]
                        for k in range(GRP))

            def pair_body(kp, carry, _grp=grp):
                packed = [plsc.load_gather(pcb_v, [carry[k]])
                          for k in range(GRP)]
                c2 = kp * 2
                for k in range(GRP):
                    lo, hi = plsc.unpack(
                        plsc.bitcast(packed[k], jnp.bfloat16),
                        format=plsc.PackFormat.INTERLEAVED)
                    ov[c2, pl.ds((_grp * GRP + k) * L, L)] = lo
                    ov[c2 + 1, pl.ds((_grp * GRP + k) * L, L)] = hi
                return tuple(v + N for v in carry)

            lax.fori_loop(0, NPAIR, pair_body, ivs, unroll=2)

    fetch_idx(0, 0)
    fetch_idx(1, 1)

    def gg_body(gg, carry):
        for par in range(2):
            g = gg * 2 + par
            wait_idx(par)

            @pl.when(gg > 0)
            def _():
                wait_out(par)

            compute(par)
            start_out(g, par)

            @pl.when(gg < NCHUNK // 2 - 1)
            def _():
                fetch_idx(g + 2, par)

        return carry

    lax.fori_loop(0, NCHUNK // 2, gg_body, 0)
    wait_out(0)
    wait_out(1)


def kernel(cat_encoded_wg, shape_input_features_in, label_mask, device):
    cb_u16 = jax.lax.bitcast_convert_type(
        cat_encoded_wg.astype(jnp.bfloat16), jnp.uint16)
    lo = cb_u16[:, 0::2, :].astype(jnp.uint32)
    hi = cb_u16[:, 1::2, :].astype(jnp.uint32)
    pcb = jax.lax.bitcast_convert_type(lo | (hi << 16), jnp.int32)
    pcb = pcb.reshape(NR, NPAIR * N)
    idx = label_mask.reshape(B, HW)
    return _unpool_sc(pcb, idx)


# speedup vs baseline: 1.1821x; 1.0422x over previous
import functools

import jax
import jax.numpy as jnp
from jax import lax
from jax.experimental import pallas as pl
from jax.experimental.pallas import tpu as pltpu
from jax.experimental.pallas import tpu_sc as plsc

B, C, H, W, N = 4, 96, 384, 384, 1024
HW = H * W
NC, NS, L = 2, 16, 16
CH = C // 2
NPAIR = CH // 2
NR = B * 2
RQ = H // 4
P = W
NCHUNK = RQ
NSLICE = P // L
GRP = 8

_mesh = plsc.VectorSubcoreMesh(core_axis_name="c", subcore_axis_name="s")


@functools.partial(
    pl.kernel,
    mesh=_mesh,
    compiler_params=pltpu.CompilerParams(needs_layout_passes=False),
    out_type=jax.ShapeDtypeStruct((B, C, H, W), jnp.float32),
    scratch_types=[
        pltpu.VMEM((NPAIR * N,), jnp.int32),
        pltpu.VMEM((P,), jnp.int32),
        pltpu.VMEM((P,), jnp.int32),
        pltpu.VMEM((CH, P), jnp.float32),
        pltpu.VMEM((CH, P), jnp.float32),
        pltpu.SemaphoreType.DMA,
        pltpu.SemaphoreType.DMA,
        pltpu.SemaphoreType.DMA,
        pltpu.SemaphoreType.DMA,
    ],
)
def _unpool_sc(pcb_hbm, idx_hbm, out_hbm, pcb_v, i0, i1, o0, o1,
               is0, is1, os0, os1):
    wid = lax.axis_index("s") * NC + lax.axis_index("c")
    r = wid // 4
    q = wid % 4
    b = r // 2
    coff = (r % 2) * CH
    roff = q * RQ
    idxv, outv = [i0, i1], [o0, o1]
    isem, osem = [is0, is1], [os0, os1]

    pltpu.sync_copy(pcb_hbm.at[r], pcb_v)

    def fetch_idx(g, par):
        pltpu.async_copy(idx_hbm.at[b, pl.ds((roff + g) * P, P)],
                         idxv[par], isem[par])

    def wait_idx(par):
        pltpu.make_async_copy(idx_hbm.at[b, pl.ds(0, P)],
                              idxv[par], isem[par]).wait()

    def start_out(g, par):
        pltpu.async_copy(outv[par],
                         out_hbm.at[b, pl.ds(coff, CH), roff + g],
                         osem[par])

    def wait_out(par):
        pltpu.make_async_copy(outv[par],
                              out_hbm.at[b, pl.ds(coff, CH), roff],
                              osem[par]).wait()

    def compute(par):
        iv_ref, ov = idxv[par], outv[par]
        for grp in range(NSLICE // GRP):
            ivs = tuple(iv_ref[pl.ds((grp * GRP + k) * L, L)]
                        for k in range(GRP))

            def pair_body(kp, carry, _grp=grp):
                packed = [plsc.load_gather(pcb_v, [carry[k]])
                          for k in range(GRP)]
                c2 = kp * 2
                for k in range(GRP):
                    lo, hi = plsc.unpack(
                        plsc.bitcast(packed[k], jnp.bfloat16),
                        format=plsc.PackFormat.INTERLEAVED)
                    ov[c2, pl.ds((_grp * GRP + k) * L, L)] = lo
                    ov[c2 + 1, pl.ds((_grp * GRP + k) * L, L)] = hi
                return tuple(v + N for v in carry)

            lax.fori_loop(0, NPAIR, pair_body, ivs, unroll=2)

    fetch_idx(0, 0)
    fetch_idx(1, 1)

    def gg_body(gg, carry):
        for par in range(2):
            g = gg * 2 + par
            wait_idx(par)

            @pl.when(gg > 0)
            def _():
                wait_out(par)

            compute(par)
            start_out(g, par)

            @pl.when(gg < NCHUNK // 2 - 1)
            def _():
                fetch_idx(g + 2, par)

        return carry

    lax.fori_loop(0, NCHUNK // 2, gg_body, 0)
    wait_out(0)
    wait_out(1)


def kernel(cat_encoded_wg, shape_input_features_in, label_mask, device):
    cb_u16 = jax.lax.bitcast_convert_type(
        cat_encoded_wg.astype(jnp.bfloat16), jnp.uint16)
    lo = cb_u16[:, 0::2, :].astype(jnp.uint32)
    hi = cb_u16[:, 1::2, :].astype(jnp.uint32)
    pcb = jax.lax.bitcast_convert_type(lo | (hi << 16), jnp.int32)
    pcb = pcb.reshape(NR, NPAIR * N)
    idx = label_mask.reshape(B, HW)
    return _unpool_sc(pcb, idx)


# speedup vs baseline: 1.1834x; 1.0010x over previous
import functools

import jax
import jax.numpy as jnp
from jax import lax
from jax.experimental import pallas as pl
from jax.experimental.pallas import tpu as pltpu
from jax.experimental.pallas import tpu_sc as plsc

B, C, H, W, N = 4, 96, 384, 384, 1024
HW = H * W
NC, NS, L = 2, 16, 16
CH = C // 2
NPAIR = CH // 2
NR = B * 2
RQ = H // 4
P = W
NCHUNK = RQ
NSLICE = P // L
GRP = 12

_mesh = plsc.VectorSubcoreMesh(core_axis_name="c", subcore_axis_name="s")


@functools.partial(
    pl.kernel,
    mesh=_mesh,
    compiler_params=pltpu.CompilerParams(needs_layout_passes=False),
    out_type=jax.ShapeDtypeStruct((B, C, H, W), jnp.float32),
    scratch_types=[
        pltpu.VMEM((NPAIR * N,), jnp.int32),
        pltpu.VMEM((P,), jnp.int32),
        pltpu.VMEM((P,), jnp.int32),
        pltpu.VMEM((CH, P), jnp.float32),
        pltpu.VMEM((CH, P), jnp.float32),
        pltpu.SemaphoreType.DMA,
        pltpu.SemaphoreType.DMA,
        pltpu.SemaphoreType.DMA,
        pltpu.SemaphoreType.DMA,
    ],
)
def _unpool_sc(pcb_hbm, idx_hbm, out_hbm, pcb_v, i0, i1, o0, o1,
               is0, is1, os0, os1):
    wid = lax.axis_index("s") * NC + lax.axis_index("c")
    r = wid // 4
    q = wid % 4
    b = r // 2
    coff = (r % 2) * CH
    roff = q * RQ
    idxv, outv = [i0, i1], [o0, o1]
    isem, osem = [is0, is1], [os0, os1]

    pltpu.sync_copy(pcb_hbm.at[r], pcb_v)

    def fetch_idx(g, par):
        pltpu.async_copy(idx_hbm.at[b, pl.ds((roff + g) * P, P)],
                         idxv[par], isem[par])

    def wait_idx(par):
        pltpu.make_async_copy(idx_hbm.at[b, pl.ds(0, P)],
                              idxv[par], isem[par]).wait()

    def start_out(g, par):
        pltpu.async_copy(outv[par],
                         out_hbm.at[b, pl.ds(coff, CH), roff + g],
                         osem[par])

    def wait_out(par):
        pltpu.make_async_copy(outv[par],
                              out_hbm.at[b, pl.ds(coff, CH), roff],
                              osem[par]).wait()

    def compute(par):
        iv_ref, ov = idxv[par], outv[par]
        for grp in range(NSLICE // GRP):
            ivs = tuple(iv_ref[pl.ds((grp * GRP + k) * L, L)]
                        for k in range(GRP))

            def pair_body(kp, carry, _grp=grp):
                packed = [plsc.load_gather(pcb_v, [carry[k]])
                          for k in range(GRP)]
                c2 = kp * 2
                for k in range(GRP):
                    lo, hi = plsc.unpack(
                        plsc.bitcast(packed[k], jnp.bfloat16),
                        format=plsc.PackFormat.INTERLEAVED)
                    ov[c2, pl.ds((_grp * GRP + k) * L, L)] = lo
                    ov[c2 + 1, pl.ds((_grp * GRP + k) * L, L)] = hi
                return tuple(v + N for v in carry)

            lax.fori_loop(0, NPAIR, pair_body, ivs, unroll=2)

    fetch_idx(0, 0)
    fetch_idx(1, 1)

    def gg_body(gg, carry):
        for par in range(2):
            g = gg * 2 + par
            wait_idx(par)

            @pl.when(gg > 0)
            def _():
                wait_out(par)

            compute(par)
            start_out(g, par)

            @pl.when(gg < NCHUNK // 2 - 1)
            def _():
                fetch_idx(g + 2, par)

        return carry

    lax.fori_loop(0, NCHUNK // 2, gg_body, 0)
    wait_out(0)
    wait_out(1)


def kernel(cat_encoded_wg, shape_input_features_in, label_mask, device):
    cb_u16 = jax.lax.bitcast_convert_type(
        cat_encoded_wg.astype(jnp.bfloat16), jnp.uint16)
    lo = cb_u16[:, 0::2, :].astype(jnp.uint32)
    hi = cb_u16[:, 1::2, :].astype(jnp.uint32)
    pcb = jax.lax.bitcast_convert_type(lo | (hi << 16), jnp.int32)
    pcb = pcb.reshape(NR, NPAIR * N)
    idx = label_mask.reshape(B, HW)
    return _unpool_sc(pcb, idx)
